# baseline (speedup 1.0000x reference)
import functools

import jax
import jax.numpy as jnp
from jax import lax
from jax.experimental import pallas as pl
from jax.experimental.pallas import tpu as pltpu
from jax.experimental.pallas import tpu_sc as plsc

N_NODES = 10000
N_EDGES = 320000
D_IN = 128
D_HID = 128
D_OUT = 16

NC = 2
NS = 16
NW = NC * NS
E_PER_TILE = N_EDGES // NW
K = 400
ROWS_PER_TILE = N_NODES // NS

E1_PER_TILE = N_EDGES // NS
E1_STAGE = E1_PER_TILE // 2
R1 = E1_STAGE // K

R2 = E_PER_TILE // K

_mesh = plsc.VectorSubcoreMesh(core_axis_name="c", subcore_axis_name="s")


def _pair_pipeline(tab, src_v, dst_v, bufs, acc, gsem, ssem, rounds):
    def _gd(h, r):
        return pltpu.make_async_copy(
            tab.at[src_v.at[pl.ds(r * K, K)]], bufs.at[h], gsem.at[h])

    def fire_g(h, r):
        _gd(h, r).start()

    def proc(h, r):
        _gd(h, r).wait()
        pltpu.async_copy(
            bufs.at[h], acc.at[dst_v.at[pl.ds(r * K, K)]], ssem.at[h],
            add=True)

    def drain_s(h, r):
        pltpu.make_async_copy(
            bufs.at[h], acc.at[dst_v.at[pl.ds(r * K, K)]],
            ssem.at[h]).wait()

    pairs = rounds // 2
    fire_g(0, 0)
    fire_g(1, 1)

    def body(i, carry):
        proc(0, 2 * i)
        drain_s(0, 2 * i)
        fire_g(0, 2 * i + 2)
        proc(1, 2 * i + 1)
        drain_s(1, 2 * i + 1)
        fire_g(1, 2 * i + 3)
        return carry

    lax.fori_loop(0, pairs - 1, body, 0)
    if rounds % 2 == 0:
        proc(0, 2 * (pairs - 1))
        proc(1, 2 * (pairs - 1) + 1)
        drain_s(0, 2 * (pairs - 1))
        drain_s(1, 2 * (pairs - 1) + 1)
    else:
        proc(0, 2 * pairs - 2)
        drain_s(0, 2 * pairs - 2)
        fire_g(0, 2 * pairs)
        proc(1, 2 * pairs - 1)
        drain_s(1, 2 * pairs - 1)
        proc(0, 2 * pairs)
        drain_s(0, 2 * pairs)


@functools.partial(
    pl.kernel,
    out_type=jax.ShapeDtypeStruct((NC, N_NODES, 16), jnp.float32),
    mesh=_mesh,
    compiler_params=pltpu.CompilerParams(use_tc_tiling_on_sc=False),
    scratch_types=[
        pltpu.VMEM((E_PER_TILE,), jnp.int32),
        pltpu.VMEM((K, 16), jnp.float32),
        pltpu.VMEM_SHARED((N_NODES, 16), jnp.float32),
        pltpu.SemaphoreType.DMA,
    ],
)
def _sc_degree(dst_hbm, zeros_hbm, out_hbm, dst_v, ones_v, acc, ssem):
    c = lax.axis_index("c")
    s = lax.axis_index("s")
    wid = s * NC + c
    pltpu.sync_copy(dst_hbm.at[pl.ds(wid * E_PER_TILE, E_PER_TILE)], dst_v)

    def fill(i, carry):
        ones_v[i, :] = jnp.full((16,), 1.0, jnp.float32)
        return carry

    lax.fori_loop(0, K, fill, 0)
    pltpu.sync_copy(
        zeros_hbm.at[pl.ds(s * ROWS_PER_TILE, ROWS_PER_TILE)],
        acc.at[pl.ds(s * ROWS_PER_TILE, ROWS_PER_TILE)],
    )
    plsc.subcore_barrier()

    def body(r, carry):
        pltpu.async_copy(
            ones_v, acc.at[dst_v.at[pl.ds(r * K, K)]], ssem, add=True)
        return carry

    lax.fori_loop(0, R2, body, 0)

    def dbody(r, carry):
        pltpu.make_async_copy(
            ones_v, acc.at[dst_v.at[pl.ds(r * K, K)]], ssem).wait()
        return carry

    lax.fori_loop(0, R2, dbody, 0)
    plsc.subcore_barrier()
    pltpu.sync_copy(
        acc.at[pl.ds(s * ROWS_PER_TILE, ROWS_PER_TILE)],
        out_hbm.at[c, pl.ds(s * ROWS_PER_TILE, ROWS_PER_TILE)],
    )


@functools.partial(
    pl.kernel,
    out_type=jax.ShapeDtypeStruct((NC, N_NODES, D_HID // NC), jnp.float32),
    mesh=_mesh,
    compiler_params=pltpu.CompilerParams(use_tc_tiling_on_sc=False),
    scratch_types=[
        pltpu.VMEM((E1_STAGE,), jnp.int32),
        pltpu.VMEM((E1_STAGE,), jnp.int32),
        pltpu.VMEM((2, K, D_HID // NC), jnp.float32),
        pltpu.VMEM_SHARED((N_NODES, D_HID // NC), jnp.float32),
        pltpu.SemaphoreType.DMA((2,)),
        pltpu.SemaphoreType.DMA((2,)),
    ],
)
def _sc_conv128(src_hbm, dst_hbm, z_hbm, zeros_hbm, out_hbm,
                src_v, dst_v, bufs, acc, gsem, ssem):
    c = lax.axis_index("c")
    s = lax.axis_index("s")
    tab = z_hbm.at[c]
    pltpu.sync_copy(
        zeros_hbm.at[pl.ds(s * ROWS_PER_TILE, ROWS_PER_TILE)],
        acc.at[pl.ds(s * ROWS_PER_TILE, ROWS_PER_TILE)],
    )
    plsc.subcore_barrier()
    for half in range(2):
        base = s * E1_PER_TILE + half * E1_STAGE
        pltpu.sync_copy(src_hbm.at[pl.ds(base, E1_STAGE)], src_v)
        pltpu.sync_copy(dst_hbm.at[pl.ds(base, E1_STAGE)], dst_v)
        _pair_pipeline(tab, src_v, dst_v, bufs, acc, gsem, ssem, R1)
    plsc.subcore_barrier()
    pltpu.sync_copy(
        acc.at[pl.ds(s * ROWS_PER_TILE, ROWS_PER_TILE)],
        out_hbm.at[c, pl.ds(s * ROWS_PER_TILE, ROWS_PER_TILE)],
    )


OROWS = 313


@functools.partial(
    pl.kernel,
    out_type=jax.ShapeDtypeStruct((N_NODES, D_OUT), jnp.float32),
    mesh=_mesh,
    compiler_params=pltpu.CompilerParams(use_tc_tiling_on_sc=False),
    scratch_types=[
        pltpu.VMEM((E1_PER_TILE,), jnp.int32),
        pltpu.VMEM((E1_PER_TILE,), jnp.int32),
        pltpu.VMEM((2, K, D_OUT), jnp.float32),
        pltpu.VMEM((OROWS, D_OUT), jnp.float32),
        pltpu.VMEM((OROWS, D_OUT), jnp.float32),
        pltpu.VMEM((OROWS, D_OUT), jnp.float32),
        pltpu.VMEM((1, D_OUT), jnp.float32),
        pltpu.VMEM_SHARED((N_NODES, D_OUT), jnp.float32),
        pltpu.SemaphoreType.DMA((2,)),
        pltpu.SemaphoreType.DMA((2,)),
    ],
)
def _sc_conv2_final(src_hbm, dst_hbm, z2_hbm, dis_hbm, b2_hbm, zeros_hbm,
                    out_hbm, src_v, dst_v, bufs, qbuf, zbuf, dbuf, bbuf,
                    acc, gsem, ssem):
    c = lax.axis_index("c")
    s = lax.axis_index("s")
    pltpu.sync_copy(src_hbm.at[pl.ds(s * E1_PER_TILE, E1_PER_TILE)], src_v)
    pltpu.sync_copy(dst_hbm.at[pl.ds(s * E1_PER_TILE, E1_PER_TILE)], dst_v)
    pltpu.sync_copy(
        zeros_hbm.at[pl.ds(s * ROWS_PER_TILE, ROWS_PER_TILE)],
        acc.at[pl.ds(s * ROWS_PER_TILE, ROWS_PER_TILE)],
    )
    plsc.subcore_barrier()
    _pair_pipeline(z2_hbm, src_v, dst_v, bufs, acc, gsem, ssem,
                   E1_PER_TILE // K)
    plsc.subcore_barrier()
    base = c * (N_NODES - NS * OROWS) + s * OROWS
    pltpu.sync_copy(acc.at[pl.ds(base, OROWS)], qbuf)
    pltpu.sync_copy(z2_hbm.at[pl.ds(base, OROWS)], zbuf)
    pltpu.sync_copy(dis_hbm.at[pl.ds(base, OROWS)], dbuf)
    pltpu.sync_copy(b2_hbm, bbuf)

    def fin(r, carry):
        qbuf[r, :] = (qbuf[r, :] + zbuf[r, :]) * dbuf[r, :] + bbuf[0, :]
        return carry

    lax.fori_loop(0, OROWS, fin, 0)
    pltpu.sync_copy(qbuf, out_hbm.at[pl.ds(base, OROWS)])


BLK = 1000
GRID = (N_NODES // BLK,)


def _dis_from(degp_ref):
    deg = degp_ref[0, :, 0:1] + degp_ref[1, :, 0:1] + 1.0
    return lax.rsqrt(deg)


def _lin1_body(x_ref, w_ref, degp_ref, zp_ref, z_ref):
    y = jnp.dot(x_ref[...], w_ref[...], preferred_element_type=jnp.float32)
    z = y * _dis_from(degp_ref)
    z_ref[...] = z
    zp_ref[0] = z[:, :D_HID // NC]
    zp_ref[1] = z[:, D_HID // NC:]


def _tc_lin1(feature, W1, degp):
    return pl.pallas_call(
        _lin1_body,
        grid=GRID,
        in_specs=[
            pl.BlockSpec((BLK, D_IN), lambda i: (i, 0)),
            pl.BlockSpec((D_IN, D_HID), lambda i: (0, 0)),
            pl.BlockSpec((NC, BLK, 16), lambda i: (0, i, 0)),
        ],
        out_specs=[
            pl.BlockSpec((NC, BLK, D_HID // NC), lambda i: (0, i, 0)),
            pl.BlockSpec((BLK, D_HID), lambda i: (i, 0)),
        ],
        out_shape=[
            jax.ShapeDtypeStruct((NC, N_NODES, D_HID // NC), jnp.float32),
            jax.ShapeDtypeStruct((N_NODES, D_HID), jnp.float32),
        ],
    )(feature, W1, degp)


def _mid_body(z1_ref, p_ref, degp_ref, b1_ref, w2_ref, o_ref, d_ref):
    dis = _dis_from(degp_ref)
    accv = z1_ref[...] + jnp.concatenate([p_ref[0], p_ref[1]], axis=-1)
    h = jnp.maximum(accv * dis + b1_ref[...], 0.0)
    y2 = jnp.dot(h, w2_ref[...], preferred_element_type=jnp.float32)
    o_ref[...] = y2 * dis
    d_ref[...] = jnp.broadcast_to(dis, (BLK, D_OUT))


def _tc_mid(z1, p, degp, b1, W2):
    return pl.pallas_call(
        _mid_body,
        grid=GRID,
        in_specs=[
            pl.BlockSpec((BLK, D_HID), lambda i: (i, 0)),
            pl.BlockSpec((NC, BLK, D_HID // NC), lambda i: (0, i, 0)),
            pl.BlockSpec((NC, BLK, 16), lambda i: (0, i, 0)),
            pl.BlockSpec((1, D_HID), lambda i: (0, 0)),
            pl.BlockSpec((D_HID, D_OUT), lambda i: (0, 0)),
        ],
        out_specs=[
            pl.BlockSpec((BLK, D_OUT), lambda i: (i, 0)),
            pl.BlockSpec((BLK, D_OUT), lambda i: (i, 0)),
        ],
        out_shape=[
            jax.ShapeDtypeStruct((N_NODES, D_OUT), jnp.float32),
            jax.ShapeDtypeStruct((N_NODES, D_OUT), jnp.float32),
        ],
    )(z1, p, degp, b1, W2)


def kernel(edge, feature, W1, b1, W2, b2):
    edge = edge.astype(jnp.int32)
    src1 = edge[0]
    dst1 = edge[1]
    zeros64 = jnp.zeros((N_NODES, D_HID // NC), jnp.float32)
    zeros16 = jnp.zeros((N_NODES, 16), jnp.float32)

    degp = _sc_degree(dst1, zeros16)
    zp, z1 = _tc_lin1(feature, W1, degp)
    p = _sc_conv128(src1, dst1, zp, zeros64)
    z2, dis16 = _tc_mid(z1, p, degp, b1.reshape(1, -1), W2)
    return _sc_conv2_final(src1, dst1, z2, dis16,
                           b2.reshape(1, -1), zeros16)


# speedup vs baseline: 1.0510x; 1.0510x over previous
import functools

import jax
import jax.numpy as jnp
from jax import lax
from jax.experimental import pallas as pl
from jax.experimental.pallas import tpu as pltpu
from jax.experimental.pallas import tpu_sc as plsc

N_NODES = 10000
N_EDGES = 320000
D_IN = 128
D_HID = 128
D_OUT = 16

NC = 2
NS = 16
NW = NC * NS
E_PER_TILE = N_EDGES // NW
K = 400
ROWS_PER_TILE = N_NODES // NS

E1_PER_TILE = N_EDGES // NS
E1_STAGE = E1_PER_TILE // 2
R1 = E1_STAGE // K

R2 = E_PER_TILE // K

_mesh = plsc.VectorSubcoreMesh(core_axis_name="c", subcore_axis_name="s")


def _pair_pipeline(tab, src_v, dst_v, bufs, acc, gsem, ssem, rounds, k=K):
    def _gd(h, r):
        return pltpu.make_async_copy(
            tab.at[src_v.at[pl.ds(r * k, k)]], bufs.at[h], gsem.at[h])

    def fire_g(h, r):
        _gd(h, r).start()

    def proc(h, r):
        _gd(h, r).wait()
        pltpu.async_copy(
            bufs.at[h], acc.at[dst_v.at[pl.ds(r * k, k)]], ssem.at[h],
            add=True)

    def drain_s(h, r):
        pltpu.make_async_copy(
            bufs.at[h], acc.at[dst_v.at[pl.ds(r * k, k)]],
            ssem.at[h]).wait()

    pairs = rounds // 2
    fire_g(0, 0)
    fire_g(1, 1)

    def body(i, carry):
        proc(0, 2 * i)
        drain_s(0, 2 * i)
        fire_g(0, 2 * i + 2)
        proc(1, 2 * i + 1)
        drain_s(1, 2 * i + 1)
        fire_g(1, 2 * i + 3)
        return carry

    lax.fori_loop(0, pairs - 1, body, 0)
    if rounds % 2 == 0:
        proc(0, 2 * (pairs - 1))
        proc(1, 2 * (pairs - 1) + 1)
        drain_s(0, 2 * (pairs - 1))
        drain_s(1, 2 * (pairs - 1) + 1)
    else:
        proc(0, 2 * pairs - 2)
        drain_s(0, 2 * pairs - 2)
        fire_g(0, 2 * pairs)
        proc(1, 2 * pairs - 1)
        drain_s(1, 2 * pairs - 1)
        proc(0, 2 * pairs)
        drain_s(0, 2 * pairs)


@functools.partial(
    pl.kernel,
    out_type=jax.ShapeDtypeStruct((NC, N_NODES, 16), jnp.float32),
    mesh=_mesh,
    compiler_params=pltpu.CompilerParams(use_tc_tiling_on_sc=False),
    scratch_types=[
        pltpu.VMEM((E_PER_TILE,), jnp.int32),
        pltpu.VMEM((K, 16), jnp.float32),
        pltpu.VMEM_SHARED((N_NODES, 16), jnp.float32),
        pltpu.SemaphoreType.DMA,
    ],
)
def _sc_degree(dst_hbm, zeros_hbm, out_hbm, dst_v, ones_v, acc, ssem):
    c = lax.axis_index("c")
    s = lax.axis_index("s")
    wid = s * NC + c
    pltpu.sync_copy(dst_hbm.at[pl.ds(wid * E_PER_TILE, E_PER_TILE)], dst_v)

    def fill(i, carry):
        ones_v[i, :] = jnp.full((16,), 1.0, jnp.float32)
        return carry

    lax.fori_loop(0, K, fill, 0)
    pltpu.sync_copy(
        zeros_hbm.at[pl.ds(s * ROWS_PER_TILE, ROWS_PER_TILE)],
        acc.at[pl.ds(s * ROWS_PER_TILE, ROWS_PER_TILE)],
    )
    plsc.subcore_barrier()

    def body(r, carry):
        pltpu.async_copy(
            ones_v, acc.at[dst_v.at[pl.ds(r * K, K)]], ssem, add=True)
        return carry

    lax.fori_loop(0, R2, body, 0)

    def dbody(r, carry):
        pltpu.make_async_copy(
            ones_v, acc.at[dst_v.at[pl.ds(r * K, K)]], ssem).wait()
        return carry

    lax.fori_loop(0, R2, dbody, 0)
    plsc.subcore_barrier()
    pltpu.sync_copy(
        acc.at[pl.ds(s * ROWS_PER_TILE, ROWS_PER_TILE)],
        out_hbm.at[c, pl.ds(s * ROWS_PER_TILE, ROWS_PER_TILE)],
    )


K1 = 80
R1E = E_PER_TILE // K1


@functools.partial(
    pl.kernel,
    out_type=jax.ShapeDtypeStruct((NC, N_NODES, D_HID), jnp.float32),
    mesh=_mesh,
    compiler_params=pltpu.CompilerParams(use_tc_tiling_on_sc=False),
    scratch_types=[
        pltpu.VMEM((E_PER_TILE,), jnp.int32),
        pltpu.VMEM((E_PER_TILE,), jnp.int32),
        pltpu.VMEM((2, K1, D_HID), jnp.float32),
        pltpu.VMEM_SHARED((N_NODES, D_HID), jnp.float32),
        pltpu.SemaphoreType.DMA((2,)),
        pltpu.SemaphoreType.DMA((2,)),
    ],
)
def _sc_conv128(src_hbm, dst_hbm, z_hbm, zeros_hbm, out_hbm,
                src_v, dst_v, bufs, acc, gsem, ssem):
    c = lax.axis_index("c")
    s = lax.axis_index("s")
    wid = s * NC + c
    pltpu.sync_copy(src_hbm.at[pl.ds(wid * E_PER_TILE, E_PER_TILE)], src_v)
    pltpu.sync_copy(dst_hbm.at[pl.ds(wid * E_PER_TILE, E_PER_TILE)], dst_v)
    pltpu.sync_copy(
        zeros_hbm.at[pl.ds(s * ROWS_PER_TILE, ROWS_PER_TILE)],
        acc.at[pl.ds(s * ROWS_PER_TILE, ROWS_PER_TILE)],
    )
    plsc.subcore_barrier()
    _pair_pipeline(z_hbm, src_v, dst_v, bufs, acc, gsem, ssem, R1E, k=K1)
    plsc.subcore_barrier()
    pltpu.sync_copy(
        acc.at[pl.ds(s * ROWS_PER_TILE, ROWS_PER_TILE)],
        out_hbm.at[c, pl.ds(s * ROWS_PER_TILE, ROWS_PER_TILE)],
    )


@functools.partial(
    pl.kernel,
    out_type=jax.ShapeDtypeStruct((NC, N_NODES, D_OUT), jnp.float32),
    mesh=_mesh,
    compiler_params=pltpu.CompilerParams(use_tc_tiling_on_sc=False),
    scratch_types=[
        pltpu.VMEM((E_PER_TILE,), jnp.int32),
        pltpu.VMEM((E_PER_TILE,), jnp.int32),
        pltpu.VMEM((2, K, D_OUT), jnp.float32),
        pltpu.VMEM_SHARED((N_NODES, D_OUT), jnp.float32),
        pltpu.SemaphoreType.DMA((2,)),
        pltpu.SemaphoreType.DMA((2,)),
    ],
)
def _sc_conv16(src_hbm, dst_hbm, z_hbm, zeros_hbm, out_hbm,
               src_v, dst_v, bufs, acc, gsem, ssem):
    c = lax.axis_index("c")
    s = lax.axis_index("s")
    wid = s * NC + c
    pltpu.sync_copy(src_hbm.at[pl.ds(wid * E_PER_TILE, E_PER_TILE)], src_v)
    pltpu.sync_copy(dst_hbm.at[pl.ds(wid * E_PER_TILE, E_PER_TILE)], dst_v)
    pltpu.sync_copy(
        zeros_hbm.at[pl.ds(s * ROWS_PER_TILE, ROWS_PER_TILE)],
        acc.at[pl.ds(s * ROWS_PER_TILE, ROWS_PER_TILE)],
    )
    plsc.subcore_barrier()
    _pair_pipeline(z_hbm, src_v, dst_v, bufs, acc, gsem, ssem, R2)
    plsc.subcore_barrier()
    pltpu.sync_copy(
        acc.at[pl.ds(s * ROWS_PER_TILE, ROWS_PER_TILE)],
        out_hbm.at[c, pl.ds(s * ROWS_PER_TILE, ROWS_PER_TILE)],
    )


BLK = 1000
GRID = (N_NODES // BLK,)


def _dis_from(degp_ref):
    deg = degp_ref[0, :, 0:1] + degp_ref[1, :, 0:1] + 1.0
    return lax.rsqrt(deg)


def _lin1_body(x_ref, w_ref, degp_ref, z_ref):
    y = jnp.dot(x_ref[...], w_ref[...], preferred_element_type=jnp.float32)
    z_ref[...] = y * _dis_from(degp_ref)


def _tc_lin1(feature, W1, degp):
    return pl.pallas_call(
        _lin1_body,
        grid=GRID,
        in_specs=[
            pl.BlockSpec((BLK, D_IN), lambda i: (i, 0)),
            pl.BlockSpec((D_IN, D_HID), lambda i: (0, 0)),
            pl.BlockSpec((NC, BLK, 16), lambda i: (0, i, 0)),
        ],
        out_specs=pl.BlockSpec((BLK, D_HID), lambda i: (i, 0)),
        out_shape=jax.ShapeDtypeStruct((N_NODES, D_HID), jnp.float32),
    )(feature, W1, degp)


def _mid_body(z1_ref, p_ref, degp_ref, b1_ref, w2_ref, o_ref):
    dis = _dis_from(degp_ref)
    accv = z1_ref[...] + p_ref[0] + p_ref[1]
    h = jnp.maximum(accv * dis + b1_ref[...], 0.0)
    y2 = jnp.dot(h, w2_ref[...], preferred_element_type=jnp.float32)
    o_ref[...] = y2 * dis


def _tc_mid(z1, p, degp, b1, W2):
    return pl.pallas_call(
        _mid_body,
        grid=GRID,
        in_specs=[
            pl.BlockSpec((BLK, D_HID), lambda i: (i, 0)),
            pl.BlockSpec((NC, BLK, D_HID), lambda i: (0, i, 0)),
            pl.BlockSpec((NC, BLK, 16), lambda i: (0, i, 0)),
            pl.BlockSpec((1, D_HID), lambda i: (0, 0)),
            pl.BlockSpec((D_HID, D_OUT), lambda i: (0, 0)),
        ],
        out_specs=pl.BlockSpec((BLK, D_OUT), lambda i: (i, 0)),
        out_shape=jax.ShapeDtypeStruct((N_NODES, D_OUT), jnp.float32),
    )(z1, p, degp, b1, W2)


def _final_body(z2_ref, q_ref, degp_ref, b2_ref, o_ref):
    dis = _dis_from(degp_ref)
    accv = z2_ref[...] + q_ref[0] + q_ref[1]
    o_ref[...] = accv * dis + b2_ref[...]


def _tc_final(z2, q, degp, b2):
    return pl.pallas_call(
        _final_body,
        grid=GRID,
        in_specs=[
            pl.BlockSpec((BLK, D_OUT), lambda i: (i, 0)),
            pl.BlockSpec((NC, BLK, D_OUT), lambda i: (0, i, 0)),
            pl.BlockSpec((NC, BLK, 16), lambda i: (0, i, 0)),
            pl.BlockSpec((1, D_OUT), lambda i: (0, 0)),
        ],
        out_specs=pl.BlockSpec((BLK, D_OUT), lambda i: (i, 0)),
        out_shape=jax.ShapeDtypeStruct((N_NODES, D_OUT), jnp.float32),
    )(z2, q, degp, b2)


def kernel(edge, feature, W1, b1, W2, b2):
    edge = edge.astype(jnp.int32)
    src1 = edge[0]
    dst1 = edge[1]
    zeros128 = jnp.zeros((N_NODES, D_HID), jnp.float32)
    zeros16 = jnp.zeros((N_NODES, 16), jnp.float32)

    degp = _sc_degree(dst1, zeros16)
    z1 = _tc_lin1(feature, W1, degp)
    p = _sc_conv128(src1, dst1, z1, zeros128)
    z2 = _tc_mid(z1, p, degp, b1.reshape(1, -1), W2)
    q = _sc_conv16(src1, dst1, z2, zeros16)
    return _tc_final(z2, q, degp, b2.reshape(1, -1))
